# baseline (speedup 1.0000x reference)
import functools

import jax
import jax.numpy as jnp
from jax.experimental import pallas as pl

NEIGH_HALF = 1.0
TP = 32
KX = 8
BNC = 512


def _pool_kernel(ap_ref, ep_ref, h_ref, W_sp_ref, b_sp_ref,
                 W1_ref, b1_ref, W2_ref, b2_ref, out_ref, *, nc):
    tp = h_ref.shape[0]
    e64 = W1_ref.shape[0] - h_ref.shape[1]
    W1_top = W1_ref[:e64, :]
    A = jnp.dot(W_sp_ref[...], W1_top, preferred_element_type=jnp.float32)
    base = (jnp.dot(h_ref[...], W1_ref[e64:, :],
                    preferred_element_type=jnp.float32)
            + jnp.dot(b_sp_ref[...], W1_top,
                      preferred_element_type=jnp.float32)
            + b1_ref[...])
    A8 = jnp.concatenate(
        [A, jnp.zeros((KX - 2, A.shape[1]), jnp.float32)],
        axis=0).astype(jnp.bfloat16)

    d = ap_ref[...] - ep_ref[...]
    d = jnp.where(jnp.abs(d) > NEIGH_HALF, 0.0, d).astype(jnp.bfloat16)

    pre_xy = jnp.dot(d, A8,
                     preferred_element_type=jnp.float32)
    pre1 = pre_xy.reshape(nc, tp, -1) + base[None, :, :]
    x1 = jnp.maximum(pre1, 0).astype(jnp.bfloat16).reshape(nc * tp, -1)

    for j in range(out_ref.shape[1] // BNC):
        y = jnp.dot(x1, W2_ref[:, j * BNC:(j + 1) * BNC],
                    preferred_element_type=jnp.float32)
        ymax = jnp.max(y.reshape(nc, tp, BNC), axis=0)
        out_ref[:, j * BNC:(j + 1) * BNC] = jnp.maximum(
            ymax + b2_ref[:, j * BNC:(j + 1) * BNC], 0.0)


def kernel(h_states, end_pos, rel_pos, annotated_points, W_sp, b_sp, W1, b1,
           W2, b2, seq_start_end):
    del rel_pos, seq_start_end
    h = h_states.reshape(-1, h_states.shape[-1])
    B = h.shape[0]
    NC = annotated_points.shape[0]
    BN = W2.shape[1]
    NT = B // TP
    R = NC * TP

    ap_e = jnp.broadcast_to(annotated_points[None, :, None, :],
                            (NT, NC, TP, 2)).reshape(NT * R, 2)
    ep_e = jnp.broadcast_to(end_pos.reshape(NT, 1, TP, 2),
                            (NT, NC, TP, 2)).reshape(NT * R, 2)
    ap_e = jnp.pad(ap_e, ((0, 0), (0, KX - 2)))
    ep_e = jnp.pad(ep_e, ((0, 0), (0, KX - 2)))

    full = lambda shape: pl.BlockSpec(shape, lambda i: (0, 0))
    out = pl.pallas_call(
        functools.partial(_pool_kernel, nc=NC),
        grid=(NT,),
        in_specs=[
            pl.BlockSpec((R, KX), lambda i: (i, 0)),
            pl.BlockSpec((R, KX), lambda i: (i, 0)),
            pl.BlockSpec((TP, h.shape[1]), lambda i: (i, 0)),
            full(W_sp.shape),
            full((1, b_sp.shape[0])),
            full(W1.shape),
            full((1, b1.shape[0])),
            full(W2.shape),
            full((1, b2.shape[0])),
        ],
        out_specs=pl.BlockSpec((TP, BN), lambda i: (i, 0)),
        out_shape=jax.ShapeDtypeStruct((B, BN), jnp.float32),
    )(ap_e, ep_e, h, W_sp, b_sp.reshape(1, -1), W1, b1.reshape(1, -1),
      W2.astype(jnp.bfloat16), b2.reshape(1, -1))
    return out


# speedup vs baseline: 1.5170x; 1.5170x over previous
import functools

import jax
import jax.numpy as jnp
from jax.experimental import pallas as pl

NEIGH_HALF = 1.0
TP = 16


def _pool_kernel(epx_ref, epy_ref, apx_ref, apy_ref, h_ref, W_sp_ref,
                 b_sp_ref, W1_ref, b1_ref, W2_ref, b2_ref, out_ref, *, nc):
    tp = h_ref.shape[0]
    e64 = W1_ref.shape[0] - h_ref.shape[1]
    W1_top = W1_ref[:e64, :]
    A = jnp.dot(W_sp_ref[...], W1_top, preferred_element_type=jnp.float32)
    base = (jnp.dot(h_ref[...], W1_ref[e64:, :],
                    preferred_element_type=jnp.float32)
            + jnp.dot(b_sp_ref[...], W1_top,
                      preferred_element_type=jnp.float32)
            + b1_ref[...])

    rx = apx_ref[...] - epx_ref[...]
    ry = apy_ref[...] - epy_ref[...]
    rx = jnp.where(jnp.abs(rx) > NEIGH_HALF, 0.0, rx)
    ry = jnp.where(jnp.abs(ry) > NEIGH_HALF, 0.0, ry)

    pre1 = (rx[:, :, None] * A[0][None, None, :]
            + ry[:, :, None] * A[1][None, None, :]
            + base[:, None, :])
    x1 = jnp.maximum(pre1, 0.0).astype(jnp.bfloat16).reshape(tp * nc, -1)

    y = jnp.dot(x1, W2_ref[...], preferred_element_type=jnp.float32)
    ymax = jnp.max(y.reshape(tp, nc, -1), axis=1)
    out_ref[...] = jnp.maximum(ymax + b2_ref[...], 0.0)


def kernel(h_states, end_pos, rel_pos, annotated_points, W_sp, b_sp, W1, b1,
           W2, b2, seq_start_end):
    del rel_pos, seq_start_end
    h = h_states.reshape(-1, h_states.shape[-1])
    B = h.shape[0]
    NC = annotated_points.shape[0]
    BN = W2.shape[1]

    epx = end_pos[:, 0:1]
    epy = end_pos[:, 1:2]
    apx = annotated_points[:, 0].reshape(1, NC)
    apy = annotated_points[:, 1].reshape(1, NC)

    full = lambda shape: pl.BlockSpec(shape, lambda i: (0, 0))
    out = pl.pallas_call(
        functools.partial(_pool_kernel, nc=NC),
        grid=(B // TP,),
        in_specs=[
            pl.BlockSpec((TP, 1), lambda i: (i, 0)),
            pl.BlockSpec((TP, 1), lambda i: (i, 0)),
            full((1, NC)),
            full((1, NC)),
            pl.BlockSpec((TP, h.shape[1]), lambda i: (i, 0)),
            full(W_sp.shape),
            full((1, b_sp.shape[0])),
            full(W1.shape),
            full((1, b1.shape[0])),
            full(W2.shape),
            full((1, b2.shape[0])),
        ],
        out_specs=pl.BlockSpec((TP, BN), lambda i: (i, 0)),
        out_shape=jax.ShapeDtypeStruct((B, BN), jnp.float32),
    )(epx, epy, apx, apy, h, W_sp, b_sp.reshape(1, -1), W1, b1.reshape(1, -1),
      W2.astype(jnp.bfloat16), b2.reshape(1, -1))
    return out


# speedup vs baseline: 1.5509x; 1.0223x over previous
import functools

import jax
import jax.numpy as jnp
from jax.experimental import pallas as pl

NEIGH_HALF = 1.0
TP = 16


def _pool_kernel(epx_ref, epy_ref, apx_ref, apy_ref, h_ref, W_sp_ref,
                 b_sp_ref, W1_ref, b1_ref, W2_ref, b2_ref, out_ref, *, nc):
    tp = h_ref.shape[0]
    e64 = W1_ref.shape[0] - h_ref.shape[1]
    W1_top = W1_ref[:e64, :]
    A = jnp.dot(W_sp_ref[...], W1_top, preferred_element_type=jnp.float32)
    base = (jnp.dot(h_ref[...], W1_ref[e64:, :],
                    preferred_element_type=jnp.float32)
            + jnp.dot(b_sp_ref[...], W1_top,
                      preferred_element_type=jnp.float32)
            + b1_ref[...])

    rx = apx_ref[...] - epx_ref[...]
    ry = apy_ref[...] - epy_ref[...]
    rx = jnp.where(jnp.abs(rx) > NEIGH_HALF, 0.0, rx)
    ry = jnp.where(jnp.abs(ry) > NEIGH_HALF, 0.0, ry)

    pre1 = (ry[:, :, None] * A[1][None, None, :]
            + (rx[:, :, None] * A[0][None, None, :]
               + base[:, None, :]))
    x1 = jnp.maximum(pre1, 0.0).astype(jnp.bfloat16).reshape(tp * nc, -1)

    y = jnp.dot(x1, W2_ref[...], preferred_element_type=jnp.float32)
    ymax = jnp.max(y.reshape(tp, nc, -1), axis=1)
    out_ref[...] = jnp.maximum(ymax + b2_ref[...], 0.0)


def kernel(h_states, end_pos, rel_pos, annotated_points, W_sp, b_sp, W1, b1,
           W2, b2, seq_start_end):
    del rel_pos, seq_start_end
    h = h_states.reshape(-1, h_states.shape[-1])
    B = h.shape[0]
    NC = annotated_points.shape[0]
    BN = W2.shape[1]

    epx = end_pos[:, 0:1]
    epy = end_pos[:, 1:2]
    NCP = -(-NC // 8) * 8
    apx = annotated_points[:, 0].reshape(1, NC)
    apy = annotated_points[:, 1].reshape(1, NC)
    apx = jnp.concatenate([apx, jnp.broadcast_to(apx[:, :1], (1, NCP - NC))],
                          axis=1)
    apy = jnp.concatenate([apy, jnp.broadcast_to(apy[:, :1], (1, NCP - NC))],
                          axis=1)
    NC = NCP

    full = lambda shape: pl.BlockSpec(shape, lambda i: (0, 0))
    out = pl.pallas_call(
        functools.partial(_pool_kernel, nc=NC),
        grid=(B // TP,),
        in_specs=[
            pl.BlockSpec((TP, 1), lambda i: (i, 0)),
            pl.BlockSpec((TP, 1), lambda i: (i, 0)),
            full((1, NC)),
            full((1, NC)),
            pl.BlockSpec((TP, h.shape[1]), lambda i: (i, 0)),
            full(W_sp.shape),
            full((1, b_sp.shape[0])),
            full(W1.shape),
            full((1, b1.shape[0])),
            full(W2.shape),
            full((1, b2.shape[0])),
        ],
        out_specs=pl.BlockSpec((TP, BN), lambda i: (i, 0)),
        out_shape=jax.ShapeDtypeStruct((B, BN), jnp.float32),
    )(epx, epy, apx, apy, h, W_sp, b_sp.reshape(1, -1), W1, b1.reshape(1, -1),
      W2.astype(jnp.bfloat16), b2.reshape(1, -1))
    return out


# speedup vs baseline: 1.6792x; 1.0827x over previous
import functools

import jax
import jax.numpy as jnp
from jax.experimental import pallas as pl

NEIGH_HALF = 1.0
TP = 16


def _pool_kernel(epx_ref, epy_ref, apx_ref, apy_ref, h_ref, W_sp_ref,
                 b_sp_ref, W1_ref, b1_ref, W2_ref, b2_ref, out_ref, *, nc):
    tp = h_ref.shape[0]
    e64 = W1_ref.shape[0] - h_ref.shape[1]
    W1_top = W1_ref[:e64, :]
    A = jnp.dot(W_sp_ref[...], W1_top, preferred_element_type=jnp.float32)
    base = (jnp.dot(h_ref[...], W1_ref[e64:, :],
                    preferred_element_type=jnp.float32)
            + jnp.dot(b_sp_ref[...], W1_top,
                      preferred_element_type=jnp.float32)
            + b1_ref[...])

    rx = apx_ref[...] - epx_ref[...]
    ry = apy_ref[...] - epy_ref[...]
    rx = jnp.where(jnp.abs(rx) > NEIGH_HALF, 0.0, rx)
    ry = jnp.where(jnp.abs(ry) > NEIGH_HALF, 0.0, ry)

    pre1 = (ry[:, :, None] * A[1][None, None, :]
            + (rx[:, :, None] * A[0][None, None, :]
               + base[:, None, :]))
    x1 = jnp.maximum(pre1, 0.0).reshape(tp * nc, -1)

    y = jnp.dot(x1, W2_ref[...], preferred_element_type=jnp.float32)
    ymax = jnp.max(y.reshape(tp, nc, -1), axis=1)
    out_ref[...] = jnp.maximum(ymax + b2_ref[...], 0.0)


def kernel(h_states, end_pos, rel_pos, annotated_points, W_sp, b_sp, W1, b1,
           W2, b2, seq_start_end):
    del rel_pos, seq_start_end
    h = h_states.reshape(-1, h_states.shape[-1])
    B = h.shape[0]
    NC = annotated_points.shape[0]
    BN = W2.shape[1]

    epx = end_pos[:, 0:1]
    epy = end_pos[:, 1:2]
    NCP = -(-NC // 8) * 8
    apx = annotated_points[:, 0].reshape(1, NC)
    apy = annotated_points[:, 1].reshape(1, NC)
    apx = jnp.concatenate([apx, jnp.broadcast_to(apx[:, :1], (1, NCP - NC))],
                          axis=1)
    apy = jnp.concatenate([apy, jnp.broadcast_to(apy[:, :1], (1, NCP - NC))],
                          axis=1)
    NC = NCP

    full = lambda shape: pl.BlockSpec(shape, lambda i: (0, 0))
    out = pl.pallas_call(
        functools.partial(_pool_kernel, nc=NC),
        grid=(B // TP,),
        in_specs=[
            pl.BlockSpec((TP, 1), lambda i: (i, 0)),
            pl.BlockSpec((TP, 1), lambda i: (i, 0)),
            full((1, NC)),
            full((1, NC)),
            pl.BlockSpec((TP, h.shape[1]), lambda i: (i, 0)),
            full(W_sp.shape),
            full((1, b_sp.shape[0])),
            full(W1.shape),
            full((1, b1.shape[0])),
            full(W2.shape),
            full((1, b2.shape[0])),
        ],
        out_specs=pl.BlockSpec((TP, BN), lambda i: (i, 0)),
        out_shape=jax.ShapeDtypeStruct((B, BN), jnp.float32),
    )(epx, epy, apx, apy, h, W_sp, b_sp.reshape(1, -1), W1, b1.reshape(1, -1),
      W2, b2.reshape(1, -1))
    return out


# speedup vs baseline: 1.7472x; 1.0405x over previous
import functools

import jax
import jax.numpy as jnp
from jax.experimental import pallas as pl

NEIGH_HALF = 1.0
TP = 32


def _pool_kernel(epx_ref, epy_ref, apx_ref, apy_ref, h_ref, W_sp_ref,
                 b_sp_ref, W1_ref, b1_ref, W2_ref, b2_ref, out_ref, *, nc):
    tp = h_ref.shape[0]
    e64 = W1_ref.shape[0] - h_ref.shape[1]
    W1_top = W1_ref[:e64, :]
    A = jnp.dot(W_sp_ref[...], W1_top, preferred_element_type=jnp.float32)
    base = (jnp.dot(h_ref[...], W1_ref[e64:, :],
                    preferred_element_type=jnp.float32)
            + jnp.dot(b_sp_ref[...], W1_top,
                      preferred_element_type=jnp.float32)
            + b1_ref[...])

    rx = apx_ref[...] - epx_ref[...]
    ry = apy_ref[...] - epy_ref[...]
    rx = jnp.where(jnp.abs(rx) > NEIGH_HALF, 0.0, rx)
    ry = jnp.where(jnp.abs(ry) > NEIGH_HALF, 0.0, ry)

    pre1 = (ry[:, :, None] * A[1][None, None, :]
            + (rx[:, :, None] * A[0][None, None, :]
               + base[:, None, :]))
    x1 = jnp.maximum(pre1, 0.0).reshape(tp * nc, -1)

    y = jnp.dot(x1, W2_ref[...], preferred_element_type=jnp.float32)
    ymax = jnp.max(y.reshape(tp, nc, -1), axis=1)
    out_ref[...] = jnp.maximum(ymax + b2_ref[...], 0.0)


def kernel(h_states, end_pos, rel_pos, annotated_points, W_sp, b_sp, W1, b1,
           W2, b2, seq_start_end):
    del rel_pos, seq_start_end
    h = h_states.reshape(-1, h_states.shape[-1])
    B = h.shape[0]
    NC = annotated_points.shape[0]
    BN = W2.shape[1]

    epx = end_pos[:, 0:1]
    epy = end_pos[:, 1:2]
    NCP = -(-NC // 8) * 8
    apx = annotated_points[:, 0].reshape(1, NC)
    apy = annotated_points[:, 1].reshape(1, NC)
    apx = jnp.concatenate([apx, jnp.broadcast_to(apx[:, :1], (1, NCP - NC))],
                          axis=1)
    apy = jnp.concatenate([apy, jnp.broadcast_to(apy[:, :1], (1, NCP - NC))],
                          axis=1)
    NC = NCP

    full = lambda shape: pl.BlockSpec(shape, lambda i: (0, 0))
    out = pl.pallas_call(
        functools.partial(_pool_kernel, nc=NC),
        grid=(B // TP,),
        in_specs=[
            pl.BlockSpec((TP, 1), lambda i: (i, 0)),
            pl.BlockSpec((TP, 1), lambda i: (i, 0)),
            full((1, NC)),
            full((1, NC)),
            pl.BlockSpec((TP, h.shape[1]), lambda i: (i, 0)),
            full(W_sp.shape),
            full((1, b_sp.shape[0])),
            full(W1.shape),
            full((1, b1.shape[0])),
            full(W2.shape),
            full((1, b2.shape[0])),
        ],
        out_specs=pl.BlockSpec((TP, BN), lambda i: (i, 0)),
        out_shape=jax.ShapeDtypeStruct((B, BN), jnp.float32),
    )(epx, epy, apx, apy, h, W_sp, b_sp.reshape(1, -1), W1, b1.reshape(1, -1),
      W2, b2.reshape(1, -1))
    return out


# speedup vs baseline: 1.7886x; 1.0237x over previous
import functools

import jax
import jax.numpy as jnp
from jax.experimental import pallas as pl

NEIGH_HALF = 1.0
TP = 64


def _pool_kernel(epx_ref, epy_ref, apx_ref, apy_ref, h_ref, W_sp_ref,
                 b_sp_ref, W1_ref, b1_ref, W2_ref, b2_ref, out_ref, *, nc):
    tp = h_ref.shape[0]
    e64 = W1_ref.shape[0] - h_ref.shape[1]
    W1_top = W1_ref[:e64, :]
    A = jnp.dot(W_sp_ref[...], W1_top, preferred_element_type=jnp.float32)
    base = (jnp.dot(h_ref[...], W1_ref[e64:, :],
                    preferred_element_type=jnp.float32)
            + jnp.dot(b_sp_ref[...], W1_top,
                      preferred_element_type=jnp.float32)
            + b1_ref[...])

    rx = apx_ref[...] - epx_ref[...]
    ry = apy_ref[...] - epy_ref[...]
    rx = jnp.where(jnp.abs(rx) > NEIGH_HALF, 0.0, rx)
    ry = jnp.where(jnp.abs(ry) > NEIGH_HALF, 0.0, ry)

    pre1 = (ry[:, :, None] * A[1][None, None, :]
            + (rx[:, :, None] * A[0][None, None, :]
               + base[:, None, :]))
    x1 = jnp.maximum(pre1, 0.0).reshape(tp * nc, -1)

    y = jnp.dot(x1, W2_ref[...], preferred_element_type=jnp.float32)
    ymax = jnp.max(y.reshape(tp, nc, -1), axis=1)
    out_ref[...] = jnp.maximum(ymax + b2_ref[...], 0.0)


def kernel(h_states, end_pos, rel_pos, annotated_points, W_sp, b_sp, W1, b1,
           W2, b2, seq_start_end):
    del rel_pos, seq_start_end
    h = h_states.reshape(-1, h_states.shape[-1])
    B = h.shape[0]
    NC = annotated_points.shape[0]
    BN = W2.shape[1]

    epx = end_pos[:, 0:1]
    epy = end_pos[:, 1:2]
    NCP = -(-NC // 8) * 8
    apx = annotated_points[:, 0].reshape(1, NC)
    apy = annotated_points[:, 1].reshape(1, NC)
    apx = jnp.concatenate([apx, jnp.broadcast_to(apx[:, :1], (1, NCP - NC))],
                          axis=1)
    apy = jnp.concatenate([apy, jnp.broadcast_to(apy[:, :1], (1, NCP - NC))],
                          axis=1)
    NC = NCP

    full = lambda shape: pl.BlockSpec(shape, lambda i: (0, 0))
    out = pl.pallas_call(
        functools.partial(_pool_kernel, nc=NC),
        grid=(B // TP,),
        in_specs=[
            pl.BlockSpec((TP, 1), lambda i: (i, 0)),
            pl.BlockSpec((TP, 1), lambda i: (i, 0)),
            full((1, NC)),
            full((1, NC)),
            pl.BlockSpec((TP, h.shape[1]), lambda i: (i, 0)),
            full(W_sp.shape),
            full((1, b_sp.shape[0])),
            full(W1.shape),
            full((1, b1.shape[0])),
            full(W2.shape),
            full((1, b2.shape[0])),
        ],
        out_specs=pl.BlockSpec((TP, BN), lambda i: (i, 0)),
        out_shape=jax.ShapeDtypeStruct((B, BN), jnp.float32),
    )(epx, epy, apx, apy, h, W_sp, b_sp.reshape(1, -1), W1, b1.reshape(1, -1),
      W2, b2.reshape(1, -1))
    return out


# speedup vs baseline: 1.8128x; 1.0135x over previous
import functools

import jax
import jax.numpy as jnp
from jax.experimental import pallas as pl

NEIGH_HALF = 1.0
TP = 128


def _pool_kernel(epx_ref, epy_ref, apx_ref, apy_ref, h_ref, W_sp_ref,
                 b_sp_ref, W1_ref, b1_ref, W2_ref, b2_ref, out_ref, *, nc):
    tp = h_ref.shape[0]
    e64 = W1_ref.shape[0] - h_ref.shape[1]
    W1_top = W1_ref[:e64, :]
    A = jnp.dot(W_sp_ref[...], W1_top, preferred_element_type=jnp.float32)
    base = (jnp.dot(h_ref[...], W1_ref[e64:, :],
                    preferred_element_type=jnp.float32)
            + jnp.dot(b_sp_ref[...], W1_top,
                      preferred_element_type=jnp.float32)
            + b1_ref[...])

    rx = apx_ref[...] - epx_ref[...]
    ry = apy_ref[...] - epy_ref[...]
    rx = jnp.where(jnp.abs(rx) > NEIGH_HALF, 0.0, rx)
    ry = jnp.where(jnp.abs(ry) > NEIGH_HALF, 0.0, ry)

    pre1 = (ry[:, :, None] * A[1][None, None, :]
            + (rx[:, :, None] * A[0][None, None, :]
               + base[:, None, :]))
    x1 = jnp.maximum(pre1, 0.0).reshape(tp * nc, -1)

    y = jnp.dot(x1, W2_ref[...], preferred_element_type=jnp.float32)
    ymax = jnp.max(y.reshape(tp, nc, -1), axis=1)
    out_ref[...] = jnp.maximum(ymax + b2_ref[...], 0.0)


def kernel(h_states, end_pos, rel_pos, annotated_points, W_sp, b_sp, W1, b1,
           W2, b2, seq_start_end):
    del rel_pos, seq_start_end
    h = h_states.reshape(-1, h_states.shape[-1])
    B = h.shape[0]
    NC = annotated_points.shape[0]
    BN = W2.shape[1]

    epx = end_pos[:, 0:1]
    epy = end_pos[:, 1:2]
    NCP = -(-NC // 8) * 8
    apx = annotated_points[:, 0].reshape(1, NC)
    apy = annotated_points[:, 1].reshape(1, NC)
    apx = jnp.concatenate([apx, jnp.broadcast_to(apx[:, :1], (1, NCP - NC))],
                          axis=1)
    apy = jnp.concatenate([apy, jnp.broadcast_to(apy[:, :1], (1, NCP - NC))],
                          axis=1)
    NC = NCP

    full = lambda shape: pl.BlockSpec(shape, lambda i: (0, 0))
    out = pl.pallas_call(
        functools.partial(_pool_kernel, nc=NC),
        grid=(B // TP,),
        in_specs=[
            pl.BlockSpec((TP, 1), lambda i: (i, 0)),
            pl.BlockSpec((TP, 1), lambda i: (i, 0)),
            full((1, NC)),
            full((1, NC)),
            pl.BlockSpec((TP, h.shape[1]), lambda i: (i, 0)),
            full(W_sp.shape),
            full((1, b_sp.shape[0])),
            full(W1.shape),
            full((1, b1.shape[0])),
            full(W2.shape),
            full((1, b2.shape[0])),
        ],
        out_specs=pl.BlockSpec((TP, BN), lambda i: (i, 0)),
        out_shape=jax.ShapeDtypeStruct((B, BN), jnp.float32),
    )(epx, epy, apx, apy, h, W_sp, b_sp.reshape(1, -1), W1, b1.reshape(1, -1),
      W2, b2.reshape(1, -1))
    return out
